# Initial kernel scaffold; baseline (speedup 1.0000x reference)
import jax
import jax.numpy as jnp
from jax.experimental import pallas as pl


def kernel(x, edge_index, W0, b0, Wg0, bg0, Wg1, bg1, Wout, bout):
    raise NotImplementedError("write your pallas kernel here")


# speedup vs baseline: 8.1117x; 8.1117x over previous
import functools

import jax
import jax.numpy as jnp
from jax import lax
from jax.experimental import pallas as pl
from jax.experimental.pallas import tpu as pltpu
from jax.experimental.pallas import tpu_sc as plsc

N = 10000
D = 128
D_OUT = 64
E = 320000

NC = 2
NS = 16
CH = 128
N_PAD = 10240
RPT = N_PAD // NS
E_PAD = 327680
CPT = E_PAD // (NC * NS * CH)

_MESH = plsc.VectorSubcoreMesh(core_axis_name="c", subcore_axis_name="s")


@functools.partial(
    pl.kernel,
    out_type=jax.ShapeDtypeStruct((NC * N_PAD, D), jnp.float32),
    mesh=_MESH,
    scratch_types=[
        pltpu.VMEM((CPT, CH), jnp.int32),
        pltpu.VMEM((CH, D), jnp.float32),
        pltpu.VMEM_SHARED((N_PAD, D), jnp.float32),
    ],
)
def _sc_degree(dst_h, ones_h, zeros8_h, out_h, didx, ones_v, deg_s):
    c = lax.axis_index("c")
    s = lax.axis_index("s")
    tid = c * NS + s
    pltpu.sync_copy(zeros8_h, deg_s.at[pl.ds(s * RPT, RPT)])
    pltpu.sync_copy(ones_h, ones_v)
    pltpu.sync_copy(dst_h.at[pl.ds(tid * CPT, CPT)], didx)
    plsc.subcore_barrier()

    def body(i, carry):
        pltpu.sync_copy(ones_v, deg_s.at[didx.at[i]], add=True)
        return carry

    lax.fori_loop(0, CPT, body, 0)
    plsc.subcore_barrier()
    pltpu.sync_copy(deg_s.at[pl.ds(s * RPT, RPT)],
                    out_h.at[pl.ds(c * N_PAD + s * RPT, RPT)])


@functools.partial(
    pl.kernel,
    out_type=jax.ShapeDtypeStruct((NC * N_PAD, D), jnp.float32),
    mesh=_MESH,
    scratch_types=[
        pltpu.VMEM((CPT, CH), jnp.int32),
        pltpu.VMEM((CPT, CH), jnp.int32),
        pltpu.VMEM((CH, D), jnp.float32),
        pltpu.VMEM_SHARED((N_PAD, D), jnp.float32),
        pltpu.SemaphoreType.DMA,
    ],
)
def _sc_scatter_rows(g_h, src_h, dst_h, zeros_h, out_h,
                     sidx, didx, rows, acc_s, sem):
    c = lax.axis_index("c")
    s = lax.axis_index("s")
    tid = c * NS + s
    pltpu.sync_copy(zeros_h, acc_s.at[pl.ds(s * RPT, RPT)])
    pltpu.sync_copy(src_h.at[pl.ds(tid * CPT, CPT)], sidx)
    pltpu.sync_copy(dst_h.at[pl.ds(tid * CPT, CPT)], didx)
    plsc.subcore_barrier()

    def body(i, carry):
        pltpu.async_copy(g_h.at[sidx.at[i]], rows, sem).wait()
        pltpu.sync_copy(rows, acc_s.at[didx.at[i]], add=True)
        return carry

    lax.fori_loop(0, CPT, body, 0)
    plsc.subcore_barrier()
    pltpu.sync_copy(acc_s.at[pl.ds(s * RPT, RPT)],
                    out_h.at[pl.ds(c * N_PAD + s * RPT, RPT)])


_BLK = 2000
_GRID = N // _BLK


def _tc_in_proj(x, w0t, b0r, wg0t):
    def body(x_ref, w0t_ref, b0_ref, wg0t_ref, o_ref):
        a = jnp.dot(x_ref[...], w0t_ref[...],
                    preferred_element_type=jnp.float32) + b0_ref[...]
        a = jnp.maximum(a, 0.0)
        o_ref[...] = jnp.dot(a, wg0t_ref[...],
                             preferred_element_type=jnp.float32)

    return pl.pallas_call(
        body,
        grid=(_GRID,),
        in_specs=[
            pl.BlockSpec((_BLK, D), lambda i: (i, 0)),
            pl.BlockSpec((D, D), lambda i: (0, 0)),
            pl.BlockSpec((1, D), lambda i: (0, 0)),
            pl.BlockSpec((D, D), lambda i: (0, 0)),
        ],
        out_specs=pl.BlockSpec((_BLK, D), lambda i: (i, 0)),
        out_shape=jax.ShapeDtypeStruct((N, D), jnp.float32),
    )(x, w0t, b0r, wg0t)


def _tc_scale(hw, dinvb):
    def body(h_ref, d_ref, o_ref):
        o_ref[...] = h_ref[...] * d_ref[...]

    return pl.pallas_call(
        body,
        grid=(_GRID,),
        in_specs=[pl.BlockSpec((_BLK, D), lambda i: (i, 0)),
                  pl.BlockSpec((_BLK, D), lambda i: (i, 0))],
        out_specs=pl.BlockSpec((_BLK, D), lambda i: (i, 0)),
        out_shape=jax.ShapeDtypeStruct((N, D), jnp.float32),
    )(hw, dinvb)


def _tc_mid(s0a, s0b, g0, dinvb, bg0r, wg1t):
    def body(a_ref, b_ref, g_ref, d_ref, bias_ref, w_ref, o_ref):
        h = d_ref[...] * (a_ref[...] + b_ref[...] + g_ref[...]) + bias_ref[...]
        h = jnp.maximum(h, 0.0)
        o_ref[...] = jnp.dot(h, w_ref[...],
                             preferred_element_type=jnp.float32) * d_ref[...]

    return pl.pallas_call(
        body,
        grid=(_GRID,),
        in_specs=[
            pl.BlockSpec((_BLK, D), lambda i: (i, 0)),
            pl.BlockSpec((_BLK, D), lambda i: (i, 0)),
            pl.BlockSpec((_BLK, D), lambda i: (i, 0)),
            pl.BlockSpec((_BLK, D), lambda i: (i, 0)),
            pl.BlockSpec((1, D), lambda i: (0, 0)),
            pl.BlockSpec((D, D), lambda i: (0, 0)),
        ],
        out_specs=pl.BlockSpec((_BLK, D), lambda i: (i, 0)),
        out_shape=jax.ShapeDtypeStruct((N, D), jnp.float32),
    )(s0a, s0b, g0, dinvb, bg0r, wg1t)


def _tc_out(s1a, s1b, g1, dinvb, bg1r, woutt, boutr):
    def body(a_ref, b_ref, g_ref, d_ref, bias_ref, w_ref, bo_ref, o_ref):
        h = d_ref[...] * (a_ref[...] + b_ref[...] + g_ref[...]) + bias_ref[...]
        h = jnp.maximum(h, 0.0)
        o_ref[...] = jnp.dot(h, w_ref[...],
                             preferred_element_type=jnp.float32) + bo_ref[...]

    return pl.pallas_call(
        body,
        grid=(_GRID,),
        in_specs=[
            pl.BlockSpec((_BLK, D), lambda i: (i, 0)),
            pl.BlockSpec((_BLK, D), lambda i: (i, 0)),
            pl.BlockSpec((_BLK, D), lambda i: (i, 0)),
            pl.BlockSpec((_BLK, D), lambda i: (i, 0)),
            pl.BlockSpec((1, D), lambda i: (0, 0)),
            pl.BlockSpec((D, D_OUT), lambda i: (0, 0)),
            pl.BlockSpec((1, D_OUT), lambda i: (0, 0)),
        ],
        out_specs=pl.BlockSpec((_BLK, D_OUT), lambda i: (i, 0)),
        out_shape=jax.ShapeDtypeStruct((N, D_OUT), jnp.float32),
    )(s1a, s1b, g1, dinvb, bg1r, woutt, boutr)


def kernel(x, edge_index, W0, b0, Wg0, bg0, Wg1, bg1, Wout, bout):
    src = edge_index[0]
    dst = edge_index[1]
    pad = E_PAD - E
    src_p = jnp.concatenate([src, jnp.zeros((pad,), jnp.int32)])
    dst_p = jnp.concatenate([dst, jnp.full((pad,), N_PAD - 1, jnp.int32)])
    src2d = src_p.reshape(E_PAD // CH, CH)
    dst2d = dst_p.reshape(E_PAD // CH, CH)

    zrows = jnp.zeros((RPT, D), jnp.float32)
    ones8 = jnp.ones((CH, D), jnp.float32)
    zeros8 = zrows

    w0t = W0.T
    wg0t = Wg0.T
    wg1t = Wg1.T
    woutt = Wout.T
    b0r = b0.reshape(1, D)
    bg0r = bg0.reshape(1, D)
    bg1r = bg1.reshape(1, D)
    boutr = bout.reshape(1, D_OUT)

    deg_parts = _sc_degree(dst2d, ones8, zeros8)
    hw0 = _tc_in_proj(x, w0t, b0r, wg0t)

    deg = deg_parts[:N, 0] + deg_parts[N_PAD:N_PAD + N, 0] + 1.0
    dinv = lax.rsqrt(jnp.maximum(deg, 1.0))
    dinvb = jnp.broadcast_to(dinv[:, None], (N, D))

    g0 = _tc_scale(hw0, dinvb)
    s0 = _sc_scatter_rows(g0, src2d, dst2d, zrows)
    g1 = _tc_mid(s0[:N], s0[N_PAD:N_PAD + N], g0, dinvb, bg0r, wg1t)

    s1 = _sc_scatter_rows(g1, src2d, dst2d, zrows)
    return _tc_out(s1[:N], s1[N_PAD:N_PAD + N], g1, dinvb, bg1r, woutt, boutr)


# speedup vs baseline: 8.6346x; 1.0645x over previous
import functools

import jax
import jax.numpy as jnp
from jax import lax
from jax.experimental import pallas as pl
from jax.experimental.pallas import tpu as pltpu
from jax.experimental.pallas import tpu_sc as plsc

N = 10000
D = 128
D_OUT = 64
E = 320000

NC = 2
NS = 16
CH = 128
N_PAD = 10240
RPT = N_PAD // NS
E_PAD = 327680
CPT = E_PAD // (NC * NS * CH)

_MESH = plsc.VectorSubcoreMesh(core_axis_name="c", subcore_axis_name="s")


@functools.partial(
    pl.kernel,
    out_type=jax.ShapeDtypeStruct((NC * N_PAD, D), jnp.float32),
    mesh=_MESH,
    scratch_types=[
        pltpu.VMEM((CPT, CH), jnp.int32),
        pltpu.VMEM((CH, D), jnp.float32),
        pltpu.VMEM_SHARED((N_PAD, D), jnp.float32),
    ],
)
def _sc_degree(dst_h, ones_h, zeros8_h, out_h, didx, ones_v, deg_s):
    c = lax.axis_index("c")
    s = lax.axis_index("s")
    tid = c * NS + s
    pltpu.sync_copy(zeros8_h, deg_s.at[pl.ds(s * RPT, RPT)])
    pltpu.sync_copy(ones_h, ones_v)
    pltpu.sync_copy(dst_h.at[pl.ds(tid * CPT, CPT)], didx)
    plsc.subcore_barrier()

    def body(i, carry):
        pltpu.sync_copy(ones_v, deg_s.at[didx.at[i]], add=True)
        return carry

    lax.fori_loop(0, CPT, body, 0)
    plsc.subcore_barrier()
    pltpu.sync_copy(deg_s.at[pl.ds(s * RPT, RPT)],
                    out_h.at[pl.ds(c * N_PAD + s * RPT, RPT)])


_IB = 8
_NLAP = CPT // _IB


@functools.partial(
    pl.kernel,
    out_type=jax.ShapeDtypeStruct((NC * N_PAD, D), jnp.float32),
    mesh=_MESH,
    scratch_types=(
        [pltpu.VMEM((CPT, CH), jnp.int32),
         pltpu.VMEM((2, _IB, CH), jnp.int32),
         pltpu.VMEM((CH, D), jnp.float32),
         pltpu.VMEM((CH, D), jnp.float32)]
        + [pltpu.SemaphoreType.DMA] * 5
        + [pltpu.VMEM_SHARED((N_PAD, D), jnp.float32)]
    ),
)
def _sc_scatter_rows(g_h, src_h, dst_h, zeros_h, out_h,
                     sidx, didx, r0, r1, isem, gs0, gs1, ss0, ss1, acc_s):
    rows = (r0, r1)
    gsem = (gs0, gs1)
    ssem = (ss0, ss1)
    c = lax.axis_index("c")
    s = lax.axis_index("s")
    tid = c * NS + s
    base = tid * CPT

    def idx_load(q):
        pltpu.async_copy(dst_h.at[pl.ds(base + q * _IB, _IB)],
                         didx.at[q % 2], isem)

    def idx_wait(q):
        pltpu.make_async_copy(dst_h.at[pl.ds(base + q * _IB, _IB)],
                              didx.at[q % 2], isem).wait()

    def gather(i, b):
        pltpu.async_copy(g_h.at[sidx.at[i]], rows[b], gsem[b])

    def gather_wait(i, b):
        pltpu.make_async_copy(g_h.at[sidx.at[i]], rows[b], gsem[b]).wait()

    def scatter(v, k, b):
        pltpu.async_copy(rows[b], acc_s.at[didx.at[v, k]], ssem[b], add=True)

    def scatter_wait(v, k, b):
        pltpu.make_async_copy(rows[b], acc_s.at[didx.at[v, k]], ssem[b]).wait()

    pltpu.sync_copy(zeros_h, acc_s.at[pl.ds(s * RPT, RPT)])
    pltpu.sync_copy(src_h.at[pl.ds(base, CPT)], sidx)
    pltpu.sync_copy(dst_h.at[pl.ds(base, _IB)], didx.at[0])
    plsc.subcore_barrier()
    gather(0, 0)
    idx_load(1)

    def lap(q, carry):
        v = q % 2
        for k in range(_IB):
            b = k % 2
            i = q * _IB + k
            gather_wait(i, b)
            scatter(v, k, b)
            if k < _IB - 1:
                b2 = (k + 1) % 2
                if k == 0:
                    @pl.when(q > 0)
                    def _():
                        scatter_wait(1 - v, _IB - 1, b2)

                    @pl.when(jnp.logical_and(q > 0, q < _NLAP - 1))
                    def _():
                        idx_load(q + 1)
                else:
                    scatter_wait(v, k - 1, b2)
                gather(i + 1, b2)

        @pl.when(q < _NLAP - 1)
        def _():
            idx_wait(q + 1)
            scatter_wait(v, _IB - 2, 0)
            gather((q + 1) * _IB, 0)
        return carry

    lax.fori_loop(0, _NLAP, lap, 0)
    scatter_wait((_NLAP - 1) % 2, _IB - 2, 0)
    scatter_wait((_NLAP - 1) % 2, _IB - 1, 1)
    plsc.subcore_barrier()
    pltpu.sync_copy(acc_s.at[pl.ds(s * RPT, RPT)],
                    out_h.at[pl.ds(c * N_PAD + s * RPT, RPT)])


_BLK = 2000
_GRID = N // _BLK


def _tc_in_proj(x, w0t, b0r, wg0t):
    def body(x_ref, w0t_ref, b0_ref, wg0t_ref, o_ref):
        a = jnp.dot(x_ref[...], w0t_ref[...],
                    preferred_element_type=jnp.float32) + b0_ref[...]
        a = jnp.maximum(a, 0.0)
        o_ref[...] = jnp.dot(a, wg0t_ref[...],
                             preferred_element_type=jnp.float32)

    return pl.pallas_call(
        body,
        grid=(_GRID,),
        in_specs=[
            pl.BlockSpec((_BLK, D), lambda i: (i, 0)),
            pl.BlockSpec((D, D), lambda i: (0, 0)),
            pl.BlockSpec((1, D), lambda i: (0, 0)),
            pl.BlockSpec((D, D), lambda i: (0, 0)),
        ],
        out_specs=pl.BlockSpec((_BLK, D), lambda i: (i, 0)),
        out_shape=jax.ShapeDtypeStruct((N, D), jnp.float32),
    )(x, w0t, b0r, wg0t)


def _tc_scale(hw, dinvb):
    def body(h_ref, d_ref, o_ref):
        o_ref[...] = h_ref[...] * d_ref[...]

    return pl.pallas_call(
        body,
        grid=(_GRID,),
        in_specs=[pl.BlockSpec((_BLK, D), lambda i: (i, 0)),
                  pl.BlockSpec((_BLK, D), lambda i: (i, 0))],
        out_specs=pl.BlockSpec((_BLK, D), lambda i: (i, 0)),
        out_shape=jax.ShapeDtypeStruct((N, D), jnp.float32),
    )(hw, dinvb)


def _tc_mid(s0a, s0b, g0, dinvb, bg0r, wg1t):
    def body(a_ref, b_ref, g_ref, d_ref, bias_ref, w_ref, o_ref):
        h = d_ref[...] * (a_ref[...] + b_ref[...] + g_ref[...]) + bias_ref[...]
        h = jnp.maximum(h, 0.0)
        o_ref[...] = jnp.dot(h, w_ref[...],
                             preferred_element_type=jnp.float32) * d_ref[...]

    return pl.pallas_call(
        body,
        grid=(_GRID,),
        in_specs=[
            pl.BlockSpec((_BLK, D), lambda i: (i, 0)),
            pl.BlockSpec((_BLK, D), lambda i: (i, 0)),
            pl.BlockSpec((_BLK, D), lambda i: (i, 0)),
            pl.BlockSpec((_BLK, D), lambda i: (i, 0)),
            pl.BlockSpec((1, D), lambda i: (0, 0)),
            pl.BlockSpec((D, D), lambda i: (0, 0)),
        ],
        out_specs=pl.BlockSpec((_BLK, D), lambda i: (i, 0)),
        out_shape=jax.ShapeDtypeStruct((N, D), jnp.float32),
    )(s0a, s0b, g0, dinvb, bg0r, wg1t)


def _tc_out(s1a, s1b, g1, dinvb, bg1r, woutt, boutr):
    def body(a_ref, b_ref, g_ref, d_ref, bias_ref, w_ref, bo_ref, o_ref):
        h = d_ref[...] * (a_ref[...] + b_ref[...] + g_ref[...]) + bias_ref[...]
        h = jnp.maximum(h, 0.0)
        o_ref[...] = jnp.dot(h, w_ref[...],
                             preferred_element_type=jnp.float32) + bo_ref[...]

    return pl.pallas_call(
        body,
        grid=(_GRID,),
        in_specs=[
            pl.BlockSpec((_BLK, D), lambda i: (i, 0)),
            pl.BlockSpec((_BLK, D), lambda i: (i, 0)),
            pl.BlockSpec((_BLK, D), lambda i: (i, 0)),
            pl.BlockSpec((_BLK, D), lambda i: (i, 0)),
            pl.BlockSpec((1, D), lambda i: (0, 0)),
            pl.BlockSpec((D, D_OUT), lambda i: (0, 0)),
            pl.BlockSpec((1, D_OUT), lambda i: (0, 0)),
        ],
        out_specs=pl.BlockSpec((_BLK, D_OUT), lambda i: (i, 0)),
        out_shape=jax.ShapeDtypeStruct((N, D_OUT), jnp.float32),
    )(s1a, s1b, g1, dinvb, bg1r, woutt, boutr)


def kernel(x, edge_index, W0, b0, Wg0, bg0, Wg1, bg1, Wout, bout):
    src = edge_index[0]
    dst = edge_index[1]
    pad = E_PAD - E
    src_p = jnp.concatenate([src, jnp.zeros((pad,), jnp.int32)])
    dst_p = jnp.concatenate([dst, jnp.full((pad,), N_PAD - 1, jnp.int32)])
    src2d = src_p.reshape(E_PAD // CH, CH)
    dst2d = dst_p.reshape(E_PAD // CH, CH)

    zrows = jnp.zeros((RPT, D), jnp.float32)
    ones8 = jnp.ones((CH, D), jnp.float32)
    zeros8 = zrows

    w0t = W0.T
    wg0t = Wg0.T
    wg1t = Wg1.T
    woutt = Wout.T
    b0r = b0.reshape(1, D)
    bg0r = bg0.reshape(1, D)
    bg1r = bg1.reshape(1, D)
    boutr = bout.reshape(1, D_OUT)

    deg_parts = _sc_degree(dst2d, ones8, zeros8)
    hw0 = _tc_in_proj(x, w0t, b0r, wg0t)

    deg = deg_parts[:N, 0] + deg_parts[N_PAD:N_PAD + N, 0] + 1.0
    dinv = lax.rsqrt(jnp.maximum(deg, 1.0))
    dinvb = jnp.broadcast_to(dinv[:, None], (N, D))

    g0 = _tc_scale(hw0, dinvb)
    s0 = _sc_scatter_rows(g0, src2d, dst2d, zrows)
    g1 = _tc_mid(s0[:N], s0[N_PAD:N_PAD + N], g0, dinvb, bg0r, wg1t)

    s1 = _sc_scatter_rows(g1, src2d, dst2d, zrows)
    return _tc_out(s1[:N], s1[N_PAD:N_PAD + N], g1, dinvb, bg1r, woutt, boutr)


# speedup vs baseline: 23.2262x; 2.6899x over previous
import functools

import jax
import jax.numpy as jnp
from jax import lax
from jax.experimental import pallas as pl
from jax.experimental.pallas import tpu as pltpu
from jax.experimental.pallas import tpu_sc as plsc

N = 10000
D = 128
D_OUT = 64
E = 320000

NC = 2
NS = 16
CH = 128
N_PAD = 10240
RPT = N_PAD // NS
E_PAD = 327680
CPT = E_PAD // (NC * NS * CH)

_MESH = plsc.VectorSubcoreMesh(core_axis_name="c", subcore_axis_name="s")


@functools.partial(
    pl.kernel,
    out_type=jax.ShapeDtypeStruct((NC * N_PAD, D), jnp.float32),
    mesh=_MESH,
    scratch_types=[
        pltpu.VMEM((CPT, CH), jnp.int32),
        pltpu.VMEM((CH, D), jnp.float32),
        pltpu.VMEM_SHARED((N_PAD, D), jnp.float32),
    ],
)
def _sc_degree(dst_h, ones_h, zeros8_h, out_h, didx, ones_v, deg_s):
    c = lax.axis_index("c")
    s = lax.axis_index("s")
    tid = c * NS + s
    pltpu.sync_copy(zeros8_h, deg_s.at[pl.ds(s * RPT, RPT)])
    pltpu.sync_copy(ones_h, ones_v)
    pltpu.sync_copy(dst_h.at[pl.ds(tid * CPT, CPT)], didx)
    plsc.subcore_barrier()

    def body(i, carry):
        pltpu.sync_copy(ones_v, deg_s.at[didx.at[i]], add=True)
        return carry

    lax.fori_loop(0, CPT, body, 0)
    plsc.subcore_barrier()
    pltpu.sync_copy(deg_s.at[pl.ds(s * RPT, RPT)],
                    out_h.at[pl.ds(c * N_PAD + s * RPT, RPT)])


_IB = 8
_NLAP = CPT // _IB


@functools.partial(
    pl.kernel,
    out_type=jax.ShapeDtypeStruct((NC * N_PAD, D), jnp.float32),
    mesh=_MESH,
    scratch_types=(
        [pltpu.VMEM((CPT, CH), jnp.int32),
         pltpu.VMEM((2, _IB, CH), jnp.int32),
         pltpu.VMEM((CH, D), jnp.float32),
         pltpu.VMEM((CH, D), jnp.float32)]
        + [pltpu.SemaphoreType.DMA] * 5
        + [pltpu.VMEM_SHARED((N_PAD, D), jnp.float32)]
    ),
)
def _sc_scatter_rows(g_h, src_h, dst_h, zeros_h, out_h,
                     sidx, didx, r0, r1, isem, gs0, gs1, ss0, ss1, acc_s):
    rows = (r0, r1)
    gsem = (gs0, gs1)
    ssem = (ss0, ss1)
    c = lax.axis_index("c")
    s = lax.axis_index("s")
    tid = c * NS + s
    base = tid * CPT

    def idx_load(q):
        pltpu.async_copy(dst_h.at[pl.ds(base + q * _IB, _IB)],
                         didx.at[q % 2], isem)

    def idx_wait(q):
        pltpu.make_async_copy(dst_h.at[pl.ds(base + q * _IB, _IB)],
                              didx.at[q % 2], isem).wait()

    def gather(i, b):
        pltpu.async_copy(g_h.at[sidx.at[i]], rows[b], gsem[b])

    def gather_wait(i, b):
        pltpu.make_async_copy(g_h.at[sidx.at[i]], rows[b], gsem[b]).wait()

    def scatter(v, k, b):
        pltpu.async_copy(rows[b], acc_s.at[didx.at[v, k]], ssem[b], add=True)

    def scatter_wait(v, k, b):
        pltpu.make_async_copy(rows[b], acc_s.at[didx.at[v, k]], ssem[b]).wait()

    pltpu.sync_copy(zeros_h, acc_s.at[pl.ds(s * RPT, RPT)])
    pltpu.sync_copy(src_h.at[pl.ds(base, CPT)], sidx)
    pltpu.sync_copy(dst_h.at[pl.ds(base, _IB)], didx.at[0])
    plsc.subcore_barrier()
    gather(0, 0)
    idx_load(1)

    def lap(q, carry):
        v = q % 2
        for k in range(_IB):
            b = k % 2
            i = q * _IB + k
            gather_wait(i, b)
            scatter(v, k, b)
            if k < _IB - 1:
                b2 = (k + 1) % 2
                if k == 0:
                    @pl.when(q > 0)
                    def _():
                        scatter_wait(1 - v, _IB - 1, b2)

                    @pl.when(jnp.logical_and(q > 0, q < _NLAP - 1))
                    def _():
                        idx_load(q + 1)
                else:
                    scatter_wait(v, k - 1, b2)
                gather(i + 1, b2)

        @pl.when(q < _NLAP - 1)
        def _():
            idx_wait(q + 1)
            scatter_wait(v, _IB - 2, 0)
            gather((q + 1) * _IB, 0)
        return carry

    lax.fori_loop(0, _NLAP, lap, 0)
    scatter_wait((_NLAP - 1) % 2, _IB - 2, 0)
    scatter_wait((_NLAP - 1) % 2, _IB - 1, 1)
    plsc.subcore_barrier()
    pltpu.sync_copy(acc_s.at[pl.ds(s * RPT, RPT)],
                    out_h.at[pl.ds(c * N_PAD + s * RPT, RPT)])


_BLK = 2000
_GRID = N // _BLK


def _tc_in_proj(x, w0t, b0r, wg0t):
    def body(x_ref, w0t_ref, b0_ref, wg0t_ref, o_ref):
        a = jnp.dot(x_ref[...], w0t_ref[...],
                    preferred_element_type=jnp.float32) + b0_ref[...]
        a = jnp.maximum(a, 0.0)
        o_ref[...] = jnp.dot(a, wg0t_ref[...],
                             preferred_element_type=jnp.float32)

    return pl.pallas_call(
        body,
        grid=(_GRID,),
        in_specs=[
            pl.BlockSpec((_BLK, D), lambda i: (i, 0)),
            pl.BlockSpec((D, D), lambda i: (0, 0)),
            pl.BlockSpec((1, D), lambda i: (0, 0)),
            pl.BlockSpec((D, D), lambda i: (0, 0)),
        ],
        out_specs=pl.BlockSpec((_BLK, D), lambda i: (i, 0)),
        out_shape=jax.ShapeDtypeStruct((N, D), jnp.float32),
    )(x, w0t, b0r, wg0t)


def _tc_scale(hw, dinvb):
    def body(h_ref, d_ref, o_ref):
        o_ref[...] = h_ref[...] * d_ref[...]

    return pl.pallas_call(
        body,
        grid=(_GRID,),
        in_specs=[pl.BlockSpec((_BLK, D), lambda i: (i, 0)),
                  pl.BlockSpec((_BLK, D), lambda i: (i, 0))],
        out_specs=pl.BlockSpec((_BLK, D), lambda i: (i, 0)),
        out_shape=jax.ShapeDtypeStruct((N, D), jnp.float32),
    )(hw, dinvb)


def _tc_mid(s0a, s0b, g0, dinvb, bg0r, wg1t):
    def body(a_ref, b_ref, g_ref, d_ref, bias_ref, w_ref, o_ref):
        h = d_ref[...] * (a_ref[...] + b_ref[...] + g_ref[...]) + bias_ref[...]
        h = jnp.maximum(h, 0.0)
        o_ref[...] = jnp.dot(h, w_ref[...],
                             preferred_element_type=jnp.float32) * d_ref[...]

    return pl.pallas_call(
        body,
        grid=(_GRID,),
        in_specs=[
            pl.BlockSpec((_BLK, D), lambda i: (i, 0)),
            pl.BlockSpec((_BLK, D), lambda i: (i, 0)),
            pl.BlockSpec((_BLK, D), lambda i: (i, 0)),
            pl.BlockSpec((_BLK, D), lambda i: (i, 0)),
            pl.BlockSpec((1, D), lambda i: (0, 0)),
            pl.BlockSpec((D, D), lambda i: (0, 0)),
        ],
        out_specs=pl.BlockSpec((_BLK, D), lambda i: (i, 0)),
        out_shape=jax.ShapeDtypeStruct((N, D), jnp.float32),
    )(s0a, s0b, g0, dinvb, bg0r, wg1t)


def _tc_out(s1a, s1b, g1, dinvb, bg1r, woutt, boutr):
    def body(a_ref, b_ref, g_ref, d_ref, bias_ref, w_ref, bo_ref, o_ref):
        h = d_ref[...] * (a_ref[...] + b_ref[...] + g_ref[...]) + bias_ref[...]
        h = jnp.maximum(h, 0.0)
        o_ref[...] = jnp.dot(h, w_ref[...],
                             preferred_element_type=jnp.float32) + bo_ref[...]

    return pl.pallas_call(
        body,
        grid=(_GRID,),
        in_specs=[
            pl.BlockSpec((_BLK, D), lambda i: (i, 0)),
            pl.BlockSpec((_BLK, D), lambda i: (i, 0)),
            pl.BlockSpec((_BLK, D), lambda i: (i, 0)),
            pl.BlockSpec((_BLK, D), lambda i: (i, 0)),
            pl.BlockSpec((1, D), lambda i: (0, 0)),
            pl.BlockSpec((D, D_OUT), lambda i: (0, 0)),
            pl.BlockSpec((1, D_OUT), lambda i: (0, 0)),
        ],
        out_specs=pl.BlockSpec((_BLK, D_OUT), lambda i: (i, 0)),
        out_shape=jax.ShapeDtypeStruct((N, D_OUT), jnp.float32),
    )(s1a, s1b, g1, dinvb, bg1r, woutt, boutr)


def kernel(x, edge_index, W0, b0, Wg0, bg0, Wg1, bg1, Wout, bout):
    src = edge_index[0]
    dst = edge_index[1]
    pad = E_PAD - E
    pad_ar = jnp.arange(pad, dtype=jnp.int32)
    src_p = jnp.concatenate([src, pad_ar % N])
    dst_p = jnp.concatenate([dst, N + pad_ar % (N_PAD - N)])
    src2d = src_p.reshape(E_PAD // CH, CH)
    dst2d = dst_p.reshape(E_PAD // CH, CH)

    zrows = jnp.zeros((RPT, D), jnp.float32)
    ones8 = jnp.ones((CH, D), jnp.float32)
    zeros8 = zrows

    w0t = W0.T
    wg0t = Wg0.T
    wg1t = Wg1.T
    woutt = Wout.T
    b0r = b0.reshape(1, D)
    bg0r = bg0.reshape(1, D)
    bg1r = bg1.reshape(1, D)
    boutr = bout.reshape(1, D_OUT)

    deg_parts = _sc_degree(dst2d, ones8, zeros8)
    hw0 = _tc_in_proj(x, w0t, b0r, wg0t)

    deg = deg_parts[:N, 0] + deg_parts[N_PAD:N_PAD + N, 0] + 1.0
    dinv = lax.rsqrt(jnp.maximum(deg, 1.0))
    dinvb = jnp.broadcast_to(dinv[:, None], (N, D))

    g0 = _tc_scale(hw0, dinvb)
    s0 = _sc_scatter_rows(g0, src2d, dst2d, zrows)
    g1 = _tc_mid(s0[:N], s0[N_PAD:N_PAD + N], g0, dinvb, bg0r, wg1t)

    s1 = _sc_scatter_rows(g1, src2d, dst2d, zrows)
    return _tc_out(s1[:N], s1[N_PAD:N_PAD + N], g1, dinvb, bg1r, woutt, boutr)
